# initial kernel scaffold (unmeasured)
import functools

import jax
import jax.numpy as jnp
from jax import lax
from jax.experimental import pallas as pl
from jax.experimental.pallas import tpu as pltpu

N_DEV = 16


def kernel(x, w_mat, scale_x, scale_w):
    m_total, k_blk = x.shape
    _, n = w_mat.shape
    m_blk = m_total // N_DEV

    def body(x_ref, w_ref, sx_ref, sw_ref, out_ref,
             wbuf, xtiles, wsend, wrecv, xsend, xrecv):
        me = lax.axis_index("i")
        right = lax.rem(me + 1, N_DEV)

        def mod(v):
            return lax.rem(v + N_DEV, N_DEV)

        barrier = pltpu.get_barrier_semaphore()
        for k in range(1, N_DEV):
            pl.semaphore_signal(barrier, inc=1, device_id=(mod(me + k),),
                                device_id_type=pl.DeviceIdType.MESH)
        pl.semaphore_wait(barrier, N_DEV - 1)

        x_sends = []
        for k in range(1, N_DEV):
            t = mod(me + k)
            rdma = pltpu.make_async_remote_copy(
                src_ref=x_ref.at[pl.ds(t * m_blk, m_blk), :],
                dst_ref=xtiles.at[me],
                send_sem=xsend.at[t],
                recv_sem=xrecv.at[me],
                device_id=(t,),
                device_id_type=pl.DeviceIdType.MESH,
            )
            rdma.start()
            x_sends.append(rdma)

        def wait_w_recv(origin):
            pltpu.make_async_remote_copy(
                src_ref=wbuf.at[origin], dst_ref=wbuf.at[origin],
                send_sem=wsend.at[origin], recv_sem=wrecv.at[origin],
                device_id=(right,), device_id_type=pl.DeviceIdType.MESH,
            ).wait_recv()

        def wait_x_recv(origin):
            pltpu.make_async_remote_copy(
                src_ref=x_ref.at[pl.ds(0, m_blk), :], dst_ref=xtiles.at[origin],
                send_sem=xsend.at[origin], recv_sem=xrecv.at[origin],
                device_id=(right,), device_id_type=pl.DeviceIdType.MESH,
            ).wait_recv()

        def accum(origin, w_chunk, first):
            xt = xtiles[origin].astype(jnp.bfloat16)
            prod = jnp.dot(xt, w_chunk.astype(jnp.bfloat16),
                           preferred_element_type=jnp.float32)
            if first:
                out_ref[:, :] = prod
            else:
                out_ref[:, :] += prod

        w_descs = []
        for h in range(N_DEV - 1):
            o_send = mod(me - h)
            o_recv = mod(me - h - 1)
            rdma = pltpu.make_async_remote_copy(
                src_ref=(w_ref if h == 0 else wbuf.at[o_send]),
                dst_ref=wbuf.at[o_send],
                send_sem=wsend.at[o_send],
                recv_sem=wrecv.at[o_send],
                device_id=(right,),
                device_id_type=pl.DeviceIdType.MESH,
            )
            rdma.start()
            w_descs.append(rdma)

            if h == 0:
                x_own = x_ref[pl.ds(me * m_blk, m_blk), :].astype(jnp.bfloat16)
                out_ref[:, :] = jnp.dot(x_own, w_ref[:, :].astype(jnp.bfloat16),
                                        preferred_element_type=jnp.float32)

            wait_w_recv(o_recv)
            wait_x_recv(o_recv)
            accum(o_recv, wbuf[o_recv], first=False)

        scale = sx_ref[0] * sw_ref[0]
        y = out_ref[:, :] * scale
        out_ref[:, :] = y / (1.0 + jnp.exp(-jnp.clip(y, -60.0, 60.0)))

        for rdma in x_sends + w_descs:
            rdma.wait_send()

        @functools.partial(pl.run_scoped, sem2=pltpu.SemaphoreType.REGULAR)
        def _(sem2):
            for k in range(1, N_DEV):
                pl.semaphore_signal(sem2, inc=1, device_id=(mod(me + k),),
                                    device_id_type=pl.DeviceIdType.MESH)
            pl.semaphore_wait(sem2, N_DEV - 1)

    return pl.pallas_call(
        body,
        out_shape=jax.ShapeDtypeStruct((m_blk, n), jnp.float32),
        in_specs=[
            pl.BlockSpec(memory_space=pltpu.VMEM),
            pl.BlockSpec(memory_space=pltpu.VMEM),
            pl.BlockSpec(memory_space=pltpu.SMEM),
            pl.BlockSpec(memory_space=pltpu.SMEM),
        ],
        out_specs=pl.BlockSpec(memory_space=pltpu.VMEM),
        scratch_shapes=[
            pltpu.VMEM((N_DEV, k_blk, n), w_mat.dtype),
            pltpu.VMEM((N_DEV, m_blk, k_blk), x.dtype),
            pltpu.SemaphoreType.DMA((N_DEV,)),
            pltpu.SemaphoreType.DMA((N_DEV,)),
            pltpu.SemaphoreType.DMA((N_DEV,)),
            pltpu.SemaphoreType.DMA((N_DEV,)),
        ],
        compiler_params=pltpu.CompilerParams(collective_id=0),
    )(x, w_mat, scale_x, scale_w)


# baseline (device time: 427772 ns/iter reference)
import functools

import jax
import jax.numpy as jnp
from jax import lax
from jax.experimental import pallas as pl
from jax.experimental.pallas import tpu as pltpu

N_DEV = 16


def kernel(x, w_mat, scale_x, scale_w):
    m_total, k_blk = x.shape
    _, n = w_mat.shape
    m_blk = m_total // N_DEV

    x = x.astype(jnp.float8_e4m3fn)
    w_mat = w_mat.astype(jnp.float8_e5m2)

    def body(x_ref, w_ref, sx_ref, sw_ref, out_ref,
             wbuf, xtiles, wsend, wrecv, xsend, xrecv):
        me = lax.axis_index("i")
        right = lax.rem(me + 1, N_DEV)

        def mod(v):
            return lax.rem(v + N_DEV, N_DEV)

        barrier = pltpu.get_barrier_semaphore()
        for k in range(1, N_DEV):
            pl.semaphore_signal(barrier, inc=1, device_id=(mod(me + k),),
                                device_id_type=pl.DeviceIdType.MESH)
        pl.semaphore_wait(barrier, N_DEV - 1)

        x_sends = []
        for k in range(1, N_DEV):
            t = mod(me + k)
            rdma = pltpu.make_async_remote_copy(
                src_ref=x_ref.at[pl.ds(t * m_blk, m_blk), :],
                dst_ref=xtiles.at[me],
                send_sem=xsend.at[t],
                recv_sem=xrecv.at[me],
                device_id=(t,),
                device_id_type=pl.DeviceIdType.MESH,
            )
            rdma.start()
            x_sends.append(rdma)

        def wait_w_recv(origin):
            pltpu.make_async_remote_copy(
                src_ref=wbuf.at[origin], dst_ref=wbuf.at[origin],
                send_sem=wsend.at[origin], recv_sem=wrecv.at[origin],
                device_id=(right,), device_id_type=pl.DeviceIdType.MESH,
            ).wait_recv()

        def wait_x_recv(origin):
            pltpu.make_async_remote_copy(
                src_ref=x_ref.at[pl.ds(0, m_blk), :], dst_ref=xtiles.at[origin],
                send_sem=xsend.at[origin], recv_sem=xrecv.at[origin],
                device_id=(right,), device_id_type=pl.DeviceIdType.MESH,
            ).wait_recv()

        def accum(origin, w_chunk, first):
            xt = xtiles[origin].astype(jnp.bfloat16)
            prod = jnp.dot(xt, w_chunk.astype(jnp.bfloat16),
                           preferred_element_type=jnp.float32)
            if first:
                out_ref[:, :] = prod
            else:
                out_ref[:, :] += prod

        w_descs = []
        for h in range(N_DEV - 1):
            o_send = mod(me - h)
            o_recv = mod(me - h - 1)
            rdma = pltpu.make_async_remote_copy(
                src_ref=(w_ref if h == 0 else wbuf.at[o_send]),
                dst_ref=wbuf.at[o_send],
                send_sem=wsend.at[o_send],
                recv_sem=wrecv.at[o_send],
                device_id=(right,),
                device_id_type=pl.DeviceIdType.MESH,
            )
            rdma.start()
            w_descs.append(rdma)

            if h == 0:
                x_own = x_ref[pl.ds(me * m_blk, m_blk), :].astype(jnp.bfloat16)
                out_ref[:, :] = jnp.dot(x_own, w_ref[:, :].astype(jnp.bfloat16),
                                        preferred_element_type=jnp.float32)

            wait_w_recv(o_recv)
            wait_x_recv(o_recv)
            accum(o_recv, wbuf[o_recv], first=False)

        scale = sx_ref[0] * sw_ref[0]
        y = out_ref[:, :] * scale
        out_ref[:, :] = y / (1.0 + jnp.exp(-jnp.clip(y, -60.0, 60.0)))

        for rdma in x_sends + w_descs:
            rdma.wait_send()

        @functools.partial(pl.run_scoped, sem2=pltpu.SemaphoreType.REGULAR)
        def _(sem2):
            for k in range(1, N_DEV):
                pl.semaphore_signal(sem2, inc=1, device_id=(mod(me + k),),
                                    device_id_type=pl.DeviceIdType.MESH)
            pl.semaphore_wait(sem2, N_DEV - 1)

    return pl.pallas_call(
        body,
        out_shape=jax.ShapeDtypeStruct((m_blk, n), jnp.float32),
        in_specs=[
            pl.BlockSpec(memory_space=pltpu.VMEM),
            pl.BlockSpec(memory_space=pltpu.VMEM),
            pl.BlockSpec(memory_space=pltpu.SMEM),
            pl.BlockSpec(memory_space=pltpu.SMEM),
        ],
        out_specs=pl.BlockSpec(memory_space=pltpu.VMEM),
        scratch_shapes=[
            pltpu.VMEM((N_DEV, k_blk, n), w_mat.dtype),
            pltpu.VMEM((N_DEV, m_blk, k_blk), x.dtype),
            pltpu.SemaphoreType.DMA((N_DEV,)),
            pltpu.SemaphoreType.DMA((N_DEV,)),
            pltpu.SemaphoreType.DMA((N_DEV,)),
            pltpu.SemaphoreType.DMA((N_DEV,)),
        ],
        compiler_params=pltpu.CompilerParams(
            collective_id=0,
            vmem_limit_bytes=60 * 1024 * 1024,
        ),
    )(x, w_mat, scale_x, scale_w)
